# Initial kernel scaffold; baseline (speedup 1.0000x reference)
import jax
import jax.numpy as jnp
from jax.experimental import pallas as pl


def kernel(input, filter_w, weight):
    raise NotImplementedError("write your pallas kernel here")


# speedup vs baseline: 1.3609x; 1.3609x over previous
import functools
import math

import jax
import jax.numpy as jnp
from jax import lax
from jax.experimental import pallas as pl
from jax.experimental.pallas import tpu as pltpu
from jax.experimental.pallas import tpu_sc as plsc

D = 512
NROWS = 99
N = 2048 * 26
L = 16

_info = plsc.get_sparse_core_info()
NW = _info.num_cores * _info.num_subcores
PER_W = N // NW
K = 16
G = PER_W // K

import numpy as np

_SIGN = np.int32(-2147483648)
_ABSM = np.int32(0x7FFFFFFF)
_ONEF = np.int32(0x3F800000)


def _prep_body(f_ref, w_ref, t_ref):
    f = f_ref[...]
    w = w_ref[...]
    w0 = w[:NROWS, :]
    w1 = w[1:, :]
    t_ref[...] = jnp.where(w0 != w1, f, 2.0) * w0


_prep = pl.pallas_call(
    _prep_body,
    out_shape=jax.ShapeDtypeStruct((NROWS, D), jnp.float32),
)

_mesh = plsc.VectorSubcoreMesh(core_axis_name="c", subcore_axis_name="s")


@functools.partial(
    pl.kernel,
    mesh=_mesh,
    out_type=jax.ShapeDtypeStruct((N * D,), jnp.float32),
    scratch_types=[
        pltpu.VMEM((NROWS * D,), jnp.float32),
        pltpu.VMEM((PER_W,), jnp.float32),
        pltpu.VMEM((2, K * D), jnp.float32),
        pltpu.SemaphoreType.DMA,
        pltpu.SemaphoreType.DMA,
    ],
)
def _sc_levels(t_hbm, x_hbm, out_hbm, tab_v, inp_v, out_v, sem0, sem1):
    wid = lax.axis_index("s") * _info.num_cores + lax.axis_index("c")
    base = wid * PER_W

    pltpu.sync_copy(t_hbm, tab_v)
    pltpu.sync_copy(x_hbm.at[pl.ds(base, PER_W)], inp_v)

    sign_v = jnp.full((L,), _SIGN, dtype=jnp.int32)
    zero_v = jnp.zeros((L,), dtype=jnp.int32)

    def do_group(g, buf, sem):
        @pl.when(g >= 2)
        def _wait():
            pltpu.make_async_copy(
                out_v.at[buf],
                out_hbm.at[pl.ds(base * D, K * D)],
                sem,
            ).wait()

        row0 = g * K

        xv = inp_v[pl.ds(row0, L)]
        value = xv * 99.0
        sv = jnp.minimum(value.astype(jnp.int32), 98)
        fracv = value - sv.astype(jnp.float32)
        tbv = sv * D

        for r in range(L):
            tb = tbv[r]
            fr = jnp.full((L,), fracv[r], dtype=jnp.float32)
            ob = r * D

            def chunk(c, _, tb=tb, fr=fr, ob=ob):
                t = tab_v[pl.ds(tb + c * L, L)]
                ti = lax.bitcast_convert_type(t, jnp.int32)
                a = lax.bitcast_convert_type(ti & _ABSM, jnp.float32)
                flip = jnp.where(fr > a, sign_v, zero_v)
                ob_bits = ((ti ^ flip) & _SIGN) | _ONEF
                out_v[buf, pl.ds(ob + c * L, L)] = lax.bitcast_convert_type(
                    ob_bits, jnp.float32)
                return 0

            lax.fori_loop(0, D // L, chunk, 0, unroll=8)

        pltpu.make_async_copy(
            out_v.at[buf],
            out_hbm.at[pl.ds((base + row0) * D, K * D)],
            sem,
        ).start()

    def outer(gg, _):
        do_group(2 * gg, 0, sem0)
        do_group(2 * gg + 1, 1, sem1)
        return 0

    lax.fori_loop(0, G // 2, outer, 0, unroll=False)

    pltpu.make_async_copy(
        out_v.at[0], out_hbm.at[pl.ds(base * D, K * D)], sem0).wait()
    pltpu.make_async_copy(
        out_v.at[1], out_hbm.at[pl.ds(base * D, K * D)], sem1).wait()


def kernel(input, filter_w, weight):
    t = _prep(filter_w, weight)
    x = input.reshape(-1)
    out = _sc_levels(t.reshape(-1), x)
    return out.reshape(input.shape + (D,))


# speedup vs baseline: 2.0611x; 1.5145x over previous
import functools
import math

import jax
import jax.numpy as jnp
from jax import lax
from jax.experimental import pallas as pl
from jax.experimental.pallas import tpu as pltpu
from jax.experimental.pallas import tpu_sc as plsc

D = 512
NROWS = 99
N = 2048 * 26
L = 16

_info = plsc.get_sparse_core_info()
NW = _info.num_cores * _info.num_subcores
PER_W = N // NW
K = 16
G = PER_W // K

import numpy as np

_SIGN = np.int32(-2147483648)
_ABSM = np.int32(0x7FFFFFFF)
_ONEF = np.int32(0x3F800000)


def _prep_body(f_ref, w_ref, t_ref):
    f = f_ref[...]
    w = w_ref[...]
    w0 = w[:NROWS, :]
    w1 = w[1:, :]
    t_ref[...] = jnp.where(w0 != w1, f, 2.0) * w0


_prep = pl.pallas_call(
    _prep_body,
    out_shape=jax.ShapeDtypeStruct((NROWS, D), jnp.float32),
)

_mesh = plsc.VectorSubcoreMesh(core_axis_name="c", subcore_axis_name="s")


@functools.partial(
    pl.kernel,
    mesh=_mesh,
    out_type=jax.ShapeDtypeStruct((N * D,), jnp.float32),
    scratch_types=[
        pltpu.VMEM((NROWS * D,), jnp.float32),
        pltpu.VMEM((PER_W,), jnp.float32),
        pltpu.VMEM((2, K * D), jnp.float32),
        pltpu.SemaphoreType.DMA,
        pltpu.SemaphoreType.DMA,
    ],
)
def _sc_levels(t_hbm, x_hbm, out_hbm, tab_v, inp_v, out_v, sem0, sem1):
    wid = lax.axis_index("s") * _info.num_cores + lax.axis_index("c")
    base = wid * PER_W

    pltpu.sync_copy(t_hbm, tab_v)
    pltpu.sync_copy(x_hbm.at[pl.ds(base, PER_W)], inp_v)

    sign_v = jnp.full((L,), _SIGN, dtype=jnp.int32)
    zero_v = jnp.zeros((L,), dtype=jnp.int32)

    def do_group(g, buf, sem):
        @pl.when(g >= 2)
        def _wait():
            pltpu.make_async_copy(
                out_v.at[buf],
                out_hbm.at[pl.ds(base * D, K * D)],
                sem,
            ).wait()

        row0 = g * K

        xv = inp_v[pl.ds(row0, L)]
        value = xv * 99.0
        sv = jnp.minimum(value.astype(jnp.int32), 98)
        fracv = value - sv.astype(jnp.float32)
        tbv = sv * D

        for r in range(L):
            tb = tbv[r]
            fr = jnp.full((L,), fracv[r], dtype=jnp.float32)
            ob = r * D

            @plsc.parallel_loop(0, D, L, unroll=32)
            def _chunk(c, tb=tb, fr=fr, ob=ob):
                t = tab_v[pl.ds(tb + c, L)]
                ti = lax.bitcast_convert_type(t, jnp.int32)
                s1 = (ti & _SIGN) | _ONEF
                a = lax.bitcast_convert_type(ti & _ABSM, jnp.float32)
                flip = jnp.where(fr > a, sign_v, zero_v)
                out_v[buf, pl.ds(ob + c, L)] = lax.bitcast_convert_type(
                    s1 ^ flip, jnp.float32)

        pltpu.make_async_copy(
            out_v.at[buf],
            out_hbm.at[pl.ds((base + row0) * D, K * D)],
            sem,
        ).start()

    def outer(gg, _):
        do_group(2 * gg, 0, sem0)
        do_group(2 * gg + 1, 1, sem1)
        return 0

    lax.fori_loop(0, G // 2, outer, 0, unroll=False)

    pltpu.make_async_copy(
        out_v.at[0], out_hbm.at[pl.ds(base * D, K * D)], sem0).wait()
    pltpu.make_async_copy(
        out_v.at[1], out_hbm.at[pl.ds(base * D, K * D)], sem1).wait()


def kernel(input, filter_w, weight):
    t = _prep(filter_w, weight)
    x = input.reshape(-1)
    out = _sc_levels(t.reshape(-1), x)
    return out.reshape(input.shape + (D,))


# speedup vs baseline: 2.1903x; 1.0627x over previous
import functools
import math

import jax
import jax.numpy as jnp
from jax import lax
from jax.experimental import pallas as pl
from jax.experimental.pallas import tpu as pltpu
from jax.experimental.pallas import tpu_sc as plsc

D = 512
NROWS = 99
N = 2048 * 26
L = 16

_info = plsc.get_sparse_core_info()
NW = _info.num_cores * _info.num_subcores
PER_W = N // NW
K = 16
G = PER_W // K

import numpy as np

_SIGN = np.int32(-2147483648)
_ABSM = np.int32(0x7FFFFFFF)
_ONEF = np.int32(0x3F800000)


def _prep_body(f_ref, w_ref, t_ref):
    f = f_ref[...]
    w = w_ref[...]
    w0 = w[:NROWS, :]
    w1 = w[1:, :]
    t_ref[...] = jnp.where(w0 != w1, f, 2.0)


_prep = pl.pallas_call(
    _prep_body,
    out_shape=jax.ShapeDtypeStruct((NROWS, D), jnp.float32),
)

_mesh = plsc.VectorSubcoreMesh(core_axis_name="c", subcore_axis_name="s")


@functools.partial(
    pl.kernel,
    mesh=_mesh,
    out_type=jax.ShapeDtypeStruct((N * D,), jnp.float32),
    scratch_types=[
        pltpu.VMEM((NROWS * D,), jnp.float32),
        pltpu.VMEM((NROWS * D,), jnp.float32),
        pltpu.VMEM((PER_W,), jnp.float32),
        pltpu.VMEM((2, K * D), jnp.float32),
        pltpu.SemaphoreType.DMA,
        pltpu.SemaphoreType.DMA,
    ],
)
def _sc_levels(t_hbm, w_hbm, x_hbm, out_hbm, tab_v, wt_v, inp_v, out_v,
               sem0, sem1):
    wid = lax.axis_index("s") * _info.num_cores + lax.axis_index("c")
    base = wid * PER_W

    pltpu.sync_copy(t_hbm, tab_v)
    pltpu.sync_copy(w_hbm, wt_v)
    pltpu.sync_copy(x_hbm.at[pl.ds(base, PER_W)], inp_v)

    sign_v = jnp.full((L,), _SIGN, dtype=jnp.int32)

    def do_group(g, buf, sem):
        @pl.when(g >= 2)
        def _wait():
            pltpu.make_async_copy(
                out_v.at[buf],
                out_hbm.at[pl.ds(base * D, K * D)],
                sem,
            ).wait()

        row0 = g * K

        xv = inp_v[pl.ds(row0, L)]
        value = xv * 99.0
        sv = jnp.minimum(value.astype(jnp.int32), 98)
        fracv = value - sv.astype(jnp.float32)
        tbv = sv * D

        for r in range(L):
            tb = tbv[r]
            fr = jnp.full((L,), fracv[r], dtype=jnp.float32)
            ob = r * D

            @plsc.parallel_loop(0, D, L, unroll=32)
            def _chunk(c, tb=tb, fr=fr, ob=ob):
                a = tab_v[pl.ds(tb + c, L)]
                wi = lax.bitcast_convert_type(
                    wt_v[pl.ds(tb + c, L)], jnp.int32)
                wf = wi ^ sign_v
                out_v[buf, pl.ds(ob + c, L)] = lax.bitcast_convert_type(
                    jnp.where(fr > a, wf, wi), jnp.float32)

        pltpu.make_async_copy(
            out_v.at[buf],
            out_hbm.at[pl.ds((base + row0) * D, K * D)],
            sem,
        ).start()

    def outer(gg, _):
        do_group(2 * gg, 0, sem0)
        do_group(2 * gg + 1, 1, sem1)
        return 0

    lax.fori_loop(0, G // 2, outer, 0, unroll=False)

    pltpu.make_async_copy(
        out_v.at[0], out_hbm.at[pl.ds(base * D, K * D)], sem0).wait()
    pltpu.make_async_copy(
        out_v.at[1], out_hbm.at[pl.ds(base * D, K * D)], sem1).wait()


def kernel(input, filter_w, weight):
    t = _prep(filter_w, weight)
    x = input.reshape(-1)
    w99 = weight[:NROWS].reshape(-1)
    out = _sc_levels(t.reshape(-1), w99, x)
    return out.reshape(input.shape + (D,))


# speedup vs baseline: 2.7113x; 1.2379x over previous
import functools
import math

import jax
import jax.numpy as jnp
from jax import lax
from jax.experimental import pallas as pl
from jax.experimental.pallas import tpu as pltpu
from jax.experimental.pallas import tpu_sc as plsc

D = 512
NROWS = 99
N = 2048 * 26
L = 16
NB = 2048

_info = plsc.get_sparse_core_info()
NW = _info.num_cores * _info.num_subcores
PER_W = N // NW
SL = 26
NSL = PER_W // SL

import numpy as np

_SIGN = np.int32(-2147483648)
_ABSM = np.int32(0x7FFFFFFF)
_ONEF = np.int32(0x3F800000)


def _prep_body(f_ref, w_ref, t_ref):
    f = f_ref[...]
    w = w_ref[...]
    w0 = w[:NROWS, :]
    w1 = w[1:, :]
    t_ref[...] = jnp.where(w0 != w1, f, 2.0) * w0


_prep = pl.pallas_call(
    _prep_body,
    out_shape=jax.ShapeDtypeStruct((NROWS, D), jnp.float32),
)

_mesh = plsc.VectorSubcoreMesh(core_axis_name="c", subcore_axis_name="s")


@functools.partial(
    pl.kernel,
    mesh=_mesh,
    out_type=jax.ShapeDtypeStruct((NB, SL, D), jnp.float32),
    scratch_types=[
        pltpu.VMEM((NROWS * D,), jnp.float32),
        pltpu.VMEM((PER_W + L,), jnp.float32),
        pltpu.VMEM((2, SL, D), jnp.float32),
        pltpu.SemaphoreType.DMA,
        pltpu.SemaphoreType.DMA,
    ],
)
def _sc_levels(t_hbm, x_hbm, out_hbm, tab_v, inp_v, out_v, sem0, sem1):
    wid = lax.axis_index("s") * _info.num_cores + lax.axis_index("c")
    base = wid * PER_W
    nbase = wid * NSL

    pltpu.sync_copy(t_hbm, tab_v)
    pltpu.sync_copy(x_hbm.at[pl.ds(base, PER_W)], inp_v.at[pl.ds(0, PER_W)])

    sign_v = jnp.full((L,), _SIGN, dtype=jnp.int32)
    zero_v = jnp.zeros((L,), dtype=jnp.int32)

    def do_slice(g, buf, sem):
        @pl.when(g >= 2)
        def _wait():
            pltpu.make_async_copy(
                out_v.at[buf], out_hbm.at[nbase], sem).wait()

        row0 = g * SL
        for blk, nrows in ((0, L), (1, SL - L)):
            xv = inp_v[pl.ds(row0 + blk * L, L)]
            value = xv * 99.0
            sv = jnp.minimum(value.astype(jnp.int32), 98)
            fracv = value - sv.astype(jnp.float32)
            tbv = sv * D

            for r in range(nrows):
                tb = tbv[r]
                fr = jnp.full((L,), fracv[r], dtype=jnp.float32)
                orow = blk * L + r

                @plsc.parallel_loop(0, D, L, unroll=32)
                def _chunk(c, tb=tb, fr=fr, orow=orow):
                    t = tab_v[pl.ds(tb + c, L)]
                    ti = lax.bitcast_convert_type(t, jnp.int32)
                    s1 = (ti & _SIGN) | _ONEF
                    a = lax.bitcast_convert_type(ti & _ABSM, jnp.float32)
                    flip = jnp.where(fr > a, sign_v, zero_v)
                    out_v[buf, orow, pl.ds(c, L)] = lax.bitcast_convert_type(
                        s1 ^ flip, jnp.float32)

        pltpu.make_async_copy(
            out_v.at[buf], out_hbm.at[nbase + g], sem).start()

    def outer(gg, _):
        do_slice(2 * gg, 0, sem0)
        do_slice(2 * gg + 1, 1, sem1)
        return 0

    lax.fori_loop(0, NSL // 2, outer, 0, unroll=False)

    pltpu.make_async_copy(out_v.at[0], out_hbm.at[nbase], sem0).wait()
    pltpu.make_async_copy(out_v.at[1], out_hbm.at[nbase], sem1).wait()


def kernel(input, filter_w, weight):
    t = _prep(filter_w, weight)
    x = input.reshape(-1)
    out = _sc_levels(t.reshape(-1), x)
    return out.reshape(input.shape + (D,))


# speedup vs baseline: 3.7623x; 1.3876x over previous
import functools
import math

import jax
import jax.numpy as jnp
from jax import lax
from jax.experimental import pallas as pl
from jax.experimental.pallas import tpu as pltpu
from jax.experimental.pallas import tpu_sc as plsc

D = 512
NROWS = 99
N = 2048 * 26
L = 16
NB = 2048

_info = plsc.get_sparse_core_info()
NW = _info.num_cores * _info.num_subcores
PER_W = N // NW
SL = 26
NSL = PER_W // SL

import numpy as np

_SIGN = np.int32(-2147483648)
_ABSM = np.int32(0x7FFFFFFF)
_ONEF = np.int32(0x3F800000)


def _prep_body(f_ref, w_ref, t_ref):
    f = f_ref[...]
    w = w_ref[...]
    w0 = w[:NROWS, :]
    w1 = w[1:, :]
    t_ref[...] = jnp.where(w0 != w1, f, 2.0) * w0


_prep = pl.pallas_call(
    _prep_body,
    out_shape=jax.ShapeDtypeStruct((NROWS, D), jnp.float32),
)

_mesh = plsc.VectorSubcoreMesh(core_axis_name="c", subcore_axis_name="s")


@functools.partial(
    pl.kernel,
    mesh=_mesh,
    out_type=jax.ShapeDtypeStruct((NB, SL, D), jnp.float32),
    scratch_types=[
        pltpu.VMEM((NROWS * D,), jnp.float32),
        pltpu.VMEM((PER_W + L,), jnp.float32),
        pltpu.VMEM((4, SL, D), jnp.float32),
        pltpu.SemaphoreType.DMA,
        pltpu.SemaphoreType.DMA,
        pltpu.SemaphoreType.DMA,
        pltpu.SemaphoreType.DMA,
    ],
)
def _sc_levels(t_hbm, x_hbm, out_hbm, tab_v, inp_v, out_v,
               sem0, sem1, sem2, sem3):
    wid = lax.axis_index("s") * _info.num_cores + lax.axis_index("c")
    base = wid * PER_W
    nbase = wid * NSL

    pltpu.sync_copy(t_hbm, tab_v)
    pltpu.sync_copy(x_hbm.at[pl.ds(base, PER_W)], inp_v.at[pl.ds(0, PER_W)])

    sign_v = jnp.full((L,), _SIGN, dtype=jnp.int32)
    zero_v = jnp.zeros((L,), dtype=jnp.int32)

    def do_slice(g, buf, sem):
        @pl.when(g >= 4)
        def _wait():
            pltpu.make_async_copy(
                out_v.at[buf], out_hbm.at[nbase], sem).wait()

        row0 = g * SL
        for blk, nrows in ((0, L), (1, SL - L)):
            xv = inp_v[pl.ds(row0 + blk * L, L)]
            value = xv * 99.0
            sv = jnp.minimum(value.astype(jnp.int32), 98)
            fracv = value - sv.astype(jnp.float32)
            tbv = sv * D

            for r in range(nrows):
                tb = tbv[r]
                fr = jnp.full((L,), fracv[r], dtype=jnp.float32)
                orow = blk * L + r

                @plsc.parallel_loop(0, D, L, unroll=8)
                def _chunk(c, tb=tb, fr=fr, orow=orow):
                    t = tab_v[pl.ds(tb + c, L)]
                    ti = lax.bitcast_convert_type(t, jnp.int32)
                    s1 = (ti & _SIGN) | _ONEF
                    a = lax.bitcast_convert_type(ti & _ABSM, jnp.float32)
                    flip = jnp.where(fr > a, sign_v, zero_v)
                    out_v[buf, orow, pl.ds(c, L)] = lax.bitcast_convert_type(
                        s1 ^ flip, jnp.float32)

        pltpu.make_async_copy(
            out_v.at[buf], out_hbm.at[nbase + g], sem).start()

    def outer(gg, _):
        do_slice(4 * gg, 0, sem0)
        do_slice(4 * gg + 1, 1, sem1)
        do_slice(4 * gg + 2, 2, sem2)
        do_slice(4 * gg + 3, 3, sem3)
        return 0

    lax.fori_loop(0, NSL // 4, outer, 0, unroll=False)

    pltpu.make_async_copy(out_v.at[0], out_hbm.at[nbase], sem0).wait()
    pltpu.make_async_copy(out_v.at[1], out_hbm.at[nbase], sem1).wait()
    pltpu.make_async_copy(out_v.at[2], out_hbm.at[nbase], sem2).wait()
    pltpu.make_async_copy(out_v.at[3], out_hbm.at[nbase], sem3).wait()


def kernel(input, filter_w, weight):
    t = _prep(filter_w, weight)
    x = input.reshape(-1)
    out = _sc_levels(t.reshape(-1), x)
    return out.reshape(input.shape + (D,))
